# baseline (speedup 1.0000x reference)
import jax
import jax.numpy as jnp
from jax.experimental import pallas as pl
from jax.experimental.pallas import tpu as pltpu

_EPS = 1e-10
_MIN_EXP = -8
_NUM_EMB = 24
_OUT_D = 128
_EXP_D = 32
_LN10 = 2.302585092994046
_SQRT_LOG2E = 1.2011224087864498
_B = 256


def _body(xt_ref, rhs_ref, out_ref):
    x = xt_ref[...]
    s = x.shape[0]
    e = jnp.floor(jnp.log10(x + _EPS))
    m2 = (2.0 * _SQRT_LOG2E) * (x * jnp.exp(e * -_LN10))
    idx = jnp.clip(e.astype(jnp.int32) - _MIN_EXP, 0, _NUM_EMB - 1)
    mh = m2.astype(jnp.bfloat16)
    ml = (m2 - mh.astype(jnp.float32)).astype(jnp.bfloat16)
    rhs = rhs_ref[...]
    lanes24 = jax.lax.broadcasted_iota(jnp.int32, (s, _NUM_EMB), 1)
    lane128 = jax.lax.broadcasted_iota(jnp.int32, (1, _OUT_D), 1)
    ones2 = jnp.ones((s, 2), jnp.bfloat16)
    for r in range(_B):
        ic = jax.lax.broadcast_in_dim(idx[:, r], (s, _NUM_EMB), (0,))
        oh = (lanes24 == ic).astype(jnp.bfloat16)
        mhc = jax.lax.broadcast_in_dim(mh[:, r], (s, 1), (0,))
        mlc = jax.lax.broadcast_in_dim(ml[:, r], (s, 1), (0,))
        lhs = jnp.concatenate([oh, mhc, mlc, ones2], axis=1)
        d = jax.lax.dot_general(
            lhs, rhs, (((1,), (0,)), ((), ())),
            preferred_element_type=jnp.float32)
        out_ref[r] = jnp.where(lane128 < _EXP_D, d, jnp.exp2(-(d * d)))


@jax.jit
def kernel(numbers, table, q_values):
    b, s = numbers.shape
    xt = numbers.T
    qs = (2.0 * _SQRT_LOG2E) * q_values
    qh = qs.astype(jnp.bfloat16)
    ql = (qs - qh.astype(jnp.float32)).astype(jnp.bfloat16)
    rhs = jnp.zeros((_NUM_EMB + 4, _OUT_D), jnp.bfloat16)
    rhs = rhs.at[:_NUM_EMB, :_EXP_D].set(table.astype(jnp.bfloat16))
    rhs = rhs.at[_NUM_EMB, _EXP_D:].set(jnp.bfloat16(1.0))
    rhs = rhs.at[_NUM_EMB + 1, _EXP_D:].set(jnp.bfloat16(1.0))
    rhs = rhs.at[_NUM_EMB + 2, _EXP_D:].set(-qh)
    rhs = rhs.at[_NUM_EMB + 3, _EXP_D:].set(-ql)

    grid = (b // _B,)
    out = pl.pallas_call(
        _body,
        grid=grid,
        in_specs=[
            pl.BlockSpec((s, _B), lambda i: (0, i)),
            pl.BlockSpec((_NUM_EMB + 4, _OUT_D), lambda i: (0, 0)),
        ],
        out_specs=pl.BlockSpec((_B, s, _OUT_D), lambda i: (i, 0, 0)),
        out_shape=jax.ShapeDtypeStruct((b, s, _OUT_D), jnp.float32),
        compiler_params=pltpu.CompilerParams(
            dimension_semantics=("arbitrary",)),
    )(xt, rhs)
    return out


# speedup vs baseline: 1.0644x; 1.0644x over previous
import jax
import jax.numpy as jnp
from jax.experimental import pallas as pl
from jax.experimental.pallas import tpu as pltpu

_EPS = 1e-10
_MIN_EXP = -8
_NUM_EMB = 24
_OUT_D = 128
_EXP_D = 32
_LN10 = 2.302585092994046
_SQRT_LOG2E = 1.2011224087864498
_B = 256


def _body(xt_ref, rhs_ref, qs_ref, out_ref):
    x = xt_ref[...]
    s = x.shape[0]
    e = jnp.floor(jnp.log10(x + _EPS))
    m2 = (2.0 * _SQRT_LOG2E) * (x * jnp.exp(e * -_LN10))
    idx = jnp.clip(e.astype(jnp.int32) - _MIN_EXP, 0, _NUM_EMB - 1)
    rhs = rhs_ref[...]
    qs = qs_ref[...]
    lanes24 = jax.lax.broadcasted_iota(jnp.int32, (s, _NUM_EMB), 1)
    for r in range(_B):
        ic = jax.lax.broadcast_in_dim(idx[:, r], (s, _NUM_EMB), (0,))
        oh = (lanes24 == ic).astype(jnp.bfloat16)
        d = jax.lax.dot_general(
            oh, rhs, (((1,), (0,)), ((), ())),
            preferred_element_type=jnp.float32)
        mc = jax.lax.broadcast_in_dim(m2[:, r], (s, _OUT_D), (0,))
        t = mc - qs
        out_ref[r] = d + jnp.exp2(-(t * t))


@jax.jit
def kernel(numbers, table, q_values):
    b, s = numbers.shape
    xt = numbers.T
    rhs = jnp.zeros((_NUM_EMB, _OUT_D), jnp.bfloat16)
    rhs = rhs.at[:, :_EXP_D].set(table.astype(jnp.bfloat16))
    qs = jnp.concatenate(
        [jnp.full((_EXP_D,), 1e30, jnp.float32),
         (2.0 * _SQRT_LOG2E) * q_values]).reshape(1, _OUT_D)

    grid = (b // _B,)
    out = pl.pallas_call(
        _body,
        grid=grid,
        in_specs=[
            pl.BlockSpec((s, _B), lambda i: (0, i)),
            pl.BlockSpec((_NUM_EMB, _OUT_D), lambda i: (0, 0)),
            pl.BlockSpec((1, _OUT_D), lambda i: (0, 0)),
        ],
        out_specs=pl.BlockSpec((_B, s, _OUT_D), lambda i: (i, 0, 0)),
        out_shape=jax.ShapeDtypeStruct((b, s, _OUT_D), jnp.float32),
        compiler_params=pltpu.CompilerParams(
            dimension_semantics=("arbitrary",)),
    )(xt, rhs, qs)
    return out
